# baseline (speedup 1.0000x reference)
import functools

import jax
import jax.numpy as jnp
from jax import lax
from jax.experimental import pallas as pl
from jax.experimental.pallas import tpu as pltpu
from jax.experimental.pallas import tpu_sc as plsc

N = 10000
IN_DIM = 256
HID = 512
HEADS = 8
DPH = HID // HEADS
OUT = 128
MLP = 512

NP = 10240
BM = 256
NBLK = NP // BM
E_RAW = 160000
ET = 2 * E_RAW + N
NC, NS = 2, 16
CH = 128
NCHUNK = 84
PDEPTH = 2
E_TILE = CH * NCHUNK
E_SC = E_TILE * NS
EP = E_SC * NC
ROWS_T = NP // NS
ZR = 16
DUMMY_DST = N + 64

_f32 = jnp.float32


def _proj_body(x_ref, w_ref, b_ref, p_ref, q_ref,
               g0, g1, g2, g3, u_ref, r_ref):
    xb = x_ref[...]
    h = jnp.dot(xb, w_ref[...], preferred_element_type=_f32) + b_ref[...]
    s = jnp.dot(h, p_ref[...], preferred_element_type=_f32)
    e16 = jnp.exp(s)
    ub = jnp.dot(e16, q_ref[...], preferred_element_type=_f32)
    g = h * ub
    g0[...] = g[:, 0:128]
    g1[...] = g[:, 128:256]
    g2[...] = g[:, 256:384]
    g3[...] = g[:, 384:512]
    lane = lax.broadcasted_iota(jnp.int32, (BM, 16), 1)
    u_ref[...] = jnp.concatenate(
        [jnp.where(lane < HEADS, e16, 0.0), jnp.zeros((BM, 112), _f32)], axis=1)
    rh = e16[:, HEADS:]
    r_ref[...] = jnp.concatenate([rh, jnp.zeros_like(rh)], axis=1)


def _proj_bn_body(x_ref, sum_ref, sq_ref, bng_ref, bnb_ref,
                  w_ref, b_ref, p_ref, q_ref,
                  g0, g1, g2, g3, u_ref, r_ref):
    mean = sum_ref[...] / N
    var = sq_ref[...] / N - mean * mean
    scale = bng_ref[...] * lax.rsqrt(var + 1e-5)
    shift = bnb_ref[...] - mean * scale
    xb = jnp.maximum(x_ref[...] * scale + shift, 0.0)
    h = jnp.dot(xb, w_ref[...], preferred_element_type=_f32) + b_ref[...]
    s = jnp.dot(h, p_ref[...], preferred_element_type=_f32)
    e16 = jnp.exp(s)
    ub = jnp.dot(e16, q_ref[...], preferred_element_type=_f32)
    g = h * ub
    g0[...] = g[:, 0:128]
    g1[...] = g[:, 128:256]
    g2[...] = g[:, 256:384]
    g3[...] = g[:, 384:512]
    lane = lax.broadcasted_iota(jnp.int32, (BM, 16), 1)
    u_ref[...] = jnp.concatenate(
        [jnp.where(lane < HEADS, e16, 0.0), jnp.zeros((BM, 112), _f32)], axis=1)
    rh = e16[:, HEADS:]
    r_ref[...] = jnp.concatenate([rh, jnp.zeros_like(rh)], axis=1)


def _full2(shape):
    return pl.BlockSpec(shape, lambda i: (0, 0))


_PROJ_OUT_SPECS = (
    [pl.BlockSpec((BM, 128), lambda i: (i, 0))] * 5
    + [pl.BlockSpec((BM, 16), lambda i: (i, 0))]
)
_PROJ_OUT_SHAPE = (
    [jax.ShapeDtypeStruct((NP, 128), _f32)] * 5
    + [jax.ShapeDtypeStruct((NP, 16), _f32)]
)


def _proj1(xp, w, b, p, q):
    return pl.pallas_call(
        _proj_body,
        grid=(NBLK,),
        in_specs=[
            pl.BlockSpec((BM, IN_DIM), lambda i: (i, 0)),
            _full2((IN_DIM, HID)), _full2((1, HID)),
            _full2((HID, 16)), _full2((16, HID)),
        ],
        out_specs=_PROJ_OUT_SPECS,
        out_shape=_PROJ_OUT_SHAPE,
    )(xp, w, b, p, q)


def _proj2(agg, s1, q1, bng, bnb, w, b, p, q):
    return pl.pallas_call(
        _proj_bn_body,
        grid=(NBLK,),
        in_specs=[
            pl.BlockSpec((BM, HID), lambda i: (i, 0)),
            _full2((1, HID)), _full2((1, HID)),
            _full2((1, HID)), _full2((1, HID)),
            _full2((HID, HID)), _full2((1, HID)),
            _full2((HID, 16)), _full2((16, HID)),
        ],
        out_specs=_PROJ_OUT_SPECS,
        out_shape=_PROJ_OUT_SHAPE,
    )(agg, s1, q1, bng, bnb, w, b, p, q)


def _combine_body(o0, o1, o2, o3, ot, r_ref, q_ref,
                  agg_ref, sum_ref, sq_ref, acc_s, acc_q):
    i = pl.program_id(0)
    t16 = (ot[0] + ot[1])[:, :16]
    r = r_ref[...]
    f = r / (r * t16 + 1e-12)
    fb = jnp.dot(f, q_ref[...], preferred_element_type=_f32)
    g = jnp.concatenate(
        [o0[0] + o0[1], o1[0] + o1[1], o2[0] + o2[1], o3[0] + o3[1]], axis=1)
    agg = g * fb
    agg_ref[...] = agg
    rowid = i * BM + lax.broadcasted_iota(jnp.int32, (BM, 1), 0)
    am = jnp.where(rowid < N, agg, 0.0)

    @pl.when(i == 0)
    def _():
        acc_s[...] = jnp.zeros_like(acc_s)
        acc_q[...] = jnp.zeros_like(acc_q)

    acc_s[...] += jnp.sum(am, axis=0, keepdims=True)
    acc_q[...] += jnp.sum(am * am, axis=0, keepdims=True)
    sum_ref[...] = acc_s[...]
    sq_ref[...] = acc_q[...]


def _combine(o0, o1, o2, o3, ot, r, q):
    return pl.pallas_call(
        _combine_body,
        grid=(NBLK,),
        in_specs=[
            pl.BlockSpec((NC, BM, 128), lambda i: (0, i, 0)),
            pl.BlockSpec((NC, BM, 128), lambda i: (0, i, 0)),
            pl.BlockSpec((NC, BM, 128), lambda i: (0, i, 0)),
            pl.BlockSpec((NC, BM, 128), lambda i: (0, i, 0)),
            pl.BlockSpec((NC, BM, 128), lambda i: (0, i, 0)),
            pl.BlockSpec((BM, 16), lambda i: (i, 0)),
            _full2((16, HID)),
        ],
        out_specs=[
            pl.BlockSpec((BM, HID), lambda i: (i, 0)),
            _full2((1, HID)), _full2((1, HID)),
        ],
        out_shape=[
            jax.ShapeDtypeStruct((NP, HID), _f32),
            jax.ShapeDtypeStruct((1, HID), _f32),
            jax.ShapeDtypeStruct((1, HID), _f32),
        ],
        scratch_shapes=[pltpu.VMEM((1, HID), _f32), pltpu.VMEM((1, HID), _f32)],
    )(o0, o1, o2, o3, ot, r, q)


def _head_body(x_ref, sum_ref, sq_ref, bng_ref, bnb_ref,
               w1_ref, b1_ref, w2_ref, b2_ref, out_ref):
    mean = sum_ref[...] / N
    var = sq_ref[...] / N - mean * mean
    scale = bng_ref[...] * lax.rsqrt(var + 1e-5)
    shift = bnb_ref[...] - mean * scale
    xb = jnp.maximum(x_ref[...] * scale + shift, 0.0)
    h1 = jnp.maximum(
        jnp.dot(xb, w1_ref[...], preferred_element_type=_f32) + b1_ref[...], 0.0)
    out_ref[...] = (
        jnp.dot(h1, w2_ref[...], preferred_element_type=_f32) + b2_ref[...])


def _head(agg, s2, q2, bng, bnb, w1, b1, w2, b2):
    return pl.pallas_call(
        _head_body,
        grid=(NBLK,),
        in_specs=[
            pl.BlockSpec((BM, HID), lambda i: (i, 0)),
            _full2((1, HID)), _full2((1, HID)),
            _full2((1, HID)), _full2((1, HID)),
            _full2((HID, MLP)), _full2((1, MLP)),
            _full2((MLP, OUT)), _full2((1, OUT)),
        ],
        out_specs=pl.BlockSpec((BM, OUT), lambda i: (i, 0)),
        out_shape=jax.ShapeDtypeStruct((NP, OUT), _f32),
    )(agg, s2, q2, bng, bnb, w1, b1, w2, b2)


def _sc_agg(packed, g0, g1, g2, g3, u, zg):
    mesh = plsc.VectorSubcoreMesh(core_axis_name="c", subcore_axis_name="s")

    scratch = [
        pltpu.VMEM_SHARED((NP, 128), _f32),
        pltpu.VMEM((NCHUNK, CH), jnp.int32),
    ]
    scratch += [pltpu.VMEM((CH,), jnp.int32) for _ in range(2 * PDEPTH)]
    scratch += [pltpu.VMEM((CH, 128), _f32) for _ in range(PDEPTH)]
    scratch += [pltpu.VMEM((ZR, 128), _f32)]
    scratch += [pltpu.SemaphoreType.DMA for _ in range(PDEPTH)]

    @functools.partial(
        pl.kernel,
        out_type=[jax.ShapeDtypeStruct((NC, NP, 128), _f32)] * 5,
        mesh=mesh,
        scratch_types=scratch,
    )
    def k(pk_h, g0_h, g1_h, g2_h, g3_h, u_h, zg_h,
          o0, o1, o2, o3, ot_h, accg, pkv, *bufs):
        srcs = bufs[0:2 * PDEPTH:2]
        dsts = bufs[1:2 * PDEPTH:2]
        rows = bufs[2 * PDEPTH:3 * PDEPTH]
        zbg = bufs[3 * PDEPTH]
        sems = bufs[3 * PDEPTH + 1:]
        sc = lax.axis_index("c")
        tid = lax.axis_index("s")
        wid = sc * NS + tid
        pltpu.sync_copy(pk_h.at[wid], pkv)
        pltpu.sync_copy(zg_h, zbg)
        gs = [u_h, g0_h, g1_h, g2_h, g3_h]
        os_ = [ot_h, o0, o1, o2, o3]
        row0 = tid * ROWS_T

        def unpack(c, srcb, dstb):
            for j in range(CH // 16):
                pk = pkv[c, pl.ds(j * 16, 16)]
                srcb[pl.ds(j * 16, 16)] = lax.bitwise_and(pk, 16383)
                dstb[pl.ds(j * 16, 16)] = lax.shift_right_logical(pk, 14)

        for p in range(5):
            g = gs[p]
            for j in range(ROWS_T // ZR):
                pltpu.sync_copy(zbg, accg.at[pl.ds(row0 + j * ZR, ZR)])
            plsc.subcore_barrier()

            srcb0, srcb1 = srcs[0], srcs[1]
            dstb0, dstb1 = dsts[0], dsts[1]
            rows0, rows1 = rows[0], rows[1]
            gs0, gs1 = sems[0], sems[1]
            unpack(0, srcb0, dstb0)
            pltpu.async_copy(g.at[srcb0], rows0, gs0)

            def body(kk, carry):
                c = kk * 2
                unpack(c + 1, srcb1, dstb1)
                pltpu.make_async_copy(g.at[srcb1], rows0, gs0).wait()
                pltpu.async_copy(g.at[srcb1], rows1, gs1)
                pltpu.sync_copy(rows0, accg.at[dstb0], add=True)
                unpack(c + 2, srcb0, dstb0)
                pltpu.make_async_copy(g.at[srcb0], rows1, gs1).wait()
                pltpu.async_copy(g.at[srcb0], rows0, gs0)
                pltpu.sync_copy(rows1, accg.at[dstb1], add=True)
                return carry

            lax.fori_loop(0, NCHUNK // 2 - 1, body, 0)
            unpack(NCHUNK - 1, srcb1, dstb1)
            pltpu.make_async_copy(g.at[srcb1], rows0, gs0).wait()
            pltpu.async_copy(g.at[srcb1], rows1, gs1)
            pltpu.sync_copy(rows0, accg.at[dstb0], add=True)
            pltpu.make_async_copy(g.at[srcb1], rows1, gs1).wait()
            pltpu.sync_copy(rows1, accg.at[dstb1], add=True)

            plsc.subcore_barrier()
            pltpu.sync_copy(accg.at[pl.ds(row0, ROWS_T)],
                            os_[p].at[sc, pl.ds(row0, ROWS_T)])

    return k(packed.reshape(NC * NS, NCHUNK, CH), g0, g1, g2, g3, u, zg)


def _pack_attn(a_l, a_r):
    eye = jnp.eye(HEADS, dtype=_f32)
    pl_ = (a_l[:, :, None] * eye[:, None, :]).reshape(HID, HEADS)
    pr_ = (a_r[:, :, None] * eye[:, None, :]).reshape(HID, HEADS)
    return jnp.concatenate([pl_, pr_], axis=1)


def kernel(x, edge_index, lin1_w, lin1_b, a_l1, a_r1, bn1_g, bn1_b,
           lin2_w, lin2_b, a_l2, a_r2, bn2_g, bn2_b,
           head_w1, head_b1, head_w2, head_b2):
    loops = jnp.arange(N, dtype=edge_index.dtype)
    ei = jnp.concatenate(
        [edge_index, edge_index[::-1], jnp.stack([loops, loops], axis=0)],
        axis=1)
    pad = EP - ET
    src = jnp.concatenate([ei[0], jnp.zeros((pad,), jnp.int32)])
    pad_dst = DUMMY_DST + (jnp.arange(pad, dtype=jnp.int32) % 128)
    dst = jnp.concatenate([ei[1], pad_dst])
    packed = jnp.left_shift(dst, 14) | src

    xp = jnp.zeros((NP, IN_DIM), _f32).at[:N].set(x)
    p1 = _pack_attn(a_l1, a_r1)
    p2 = _pack_attn(a_l2, a_r2)
    q = jnp.concatenate(
        [jnp.repeat(jnp.eye(HEADS, dtype=_f32), DPH, axis=1),
         jnp.zeros((HEADS, HID), _f32)], axis=0)
    zg = jnp.zeros((ZR, 128), _f32)
    row = lambda v: v.reshape(1, -1)

    g0, g1, g2, g3, u1, r1 = _proj1(xp, lin1_w, row(lin1_b), p1, q)
    o0, o1, o2, o3, ot = _sc_agg(packed, g0, g1, g2, g3, u1, zg)
    agg1, s1, q1 = _combine(o0, o1, o2, o3, ot, r1, q)

    g0, g1, g2, g3, u2, r2 = _proj2(agg1, s1, q1, row(bn1_g), row(bn1_b),
                                    lin2_w, row(lin2_b), p2, q)
    o0, o1, o2, o3, ot = _sc_agg(packed, g0, g1, g2, g3, u2, zg)
    agg2, s2, q2 = _combine(o0, o1, o2, o3, ot, r2, q)

    logits = _head(agg2, s2, q2, row(bn2_g), row(bn2_b),
                   head_w1, row(head_b1), head_w2, row(head_b2))
    return logits[:N]


# speedup vs baseline: 1.9384x; 1.9384x over previous
import functools

import jax
import jax.numpy as jnp
from jax import lax
from jax.experimental import pallas as pl
from jax.experimental.pallas import tpu as pltpu
from jax.experimental.pallas import tpu_sc as plsc

N = 10000
IN_DIM = 256
HID = 512
HEADS = 8
DPH = HID // HEADS
OUT = 128
MLP = 512

NP = 10240
BM = 256
NBLK = NP // BM
E_RAW = 160000
ET = 2 * E_RAW + N
NC, NS = 2, 16
CH = 128
NCHUNK = 82
PDEPTH = 2
E_TILE = CH * NCHUNK
E_SC = E_TILE * NS
EP = E_SC * NC
ROWS_T = NP // NS
ZR = 16
DUMMY_DST = N + 64

_f32 = jnp.float32


def _proj_body(x_ref, w_ref, b_ref, p_ref, q_ref,
               g0, g1, g2, g3, u_ref, r_ref):
    xb = x_ref[...]
    h = jnp.dot(xb, w_ref[...], preferred_element_type=_f32) + b_ref[...]
    s = jnp.dot(h, p_ref[...], preferred_element_type=_f32)
    e16 = jnp.exp(s)
    ub = jnp.dot(e16, q_ref[...], preferred_element_type=_f32)
    g = h * ub
    g0[...] = g[:, 0:128]
    g1[...] = g[:, 128:256]
    g2[...] = g[:, 256:384]
    g3[...] = g[:, 384:512]
    lane = lax.broadcasted_iota(jnp.int32, (BM, 16), 1)
    u_ref[...] = jnp.concatenate(
        [jnp.where(lane < HEADS, e16, 0.0), jnp.zeros((BM, 112), _f32)], axis=1)
    rh = e16[:, HEADS:]
    r_ref[...] = jnp.concatenate([rh, jnp.zeros_like(rh)], axis=1)


def _proj_bn_body(x_ref, sum_ref, sq_ref, bng_ref, bnb_ref,
                  w_ref, b_ref, p_ref, q_ref,
                  g0, g1, g2, g3, u_ref, r_ref):
    mean = sum_ref[...] / N
    var = sq_ref[...] / N - mean * mean
    scale = bng_ref[...] * lax.rsqrt(var + 1e-5)
    shift = bnb_ref[...] - mean * scale
    xb = jnp.maximum(x_ref[...] * scale + shift, 0.0)
    h = jnp.dot(xb, w_ref[...], preferred_element_type=_f32) + b_ref[...]
    s = jnp.dot(h, p_ref[...], preferred_element_type=_f32)
    e16 = jnp.exp(s)
    ub = jnp.dot(e16, q_ref[...], preferred_element_type=_f32)
    g = h * ub
    g0[...] = g[:, 0:128]
    g1[...] = g[:, 128:256]
    g2[...] = g[:, 256:384]
    g3[...] = g[:, 384:512]
    lane = lax.broadcasted_iota(jnp.int32, (BM, 16), 1)
    u_ref[...] = jnp.concatenate(
        [jnp.where(lane < HEADS, e16, 0.0), jnp.zeros((BM, 112), _f32)], axis=1)
    rh = e16[:, HEADS:]
    r_ref[...] = jnp.concatenate([rh, jnp.zeros_like(rh)], axis=1)


def _full2(shape):
    return pl.BlockSpec(shape, lambda i: (0, 0))


_PROJ_OUT_SPECS = (
    [pl.BlockSpec((BM, 128), lambda i: (i, 0))] * 5
    + [pl.BlockSpec((BM, 16), lambda i: (i, 0))]
)
_PROJ_OUT_SHAPE = (
    [jax.ShapeDtypeStruct((NP, 128), _f32)] * 5
    + [jax.ShapeDtypeStruct((NP, 16), _f32)]
)


def _proj1(xp, w, b, p, q):
    return pl.pallas_call(
        _proj_body,
        grid=(NBLK,),
        in_specs=[
            pl.BlockSpec((BM, IN_DIM), lambda i: (i, 0)),
            _full2((IN_DIM, HID)), _full2((1, HID)),
            _full2((HID, 16)), _full2((16, HID)),
        ],
        out_specs=_PROJ_OUT_SPECS,
        out_shape=_PROJ_OUT_SHAPE,
    )(xp, w, b, p, q)


def _proj2(agg, s1, q1, bng, bnb, w, b, p, q):
    return pl.pallas_call(
        _proj_bn_body,
        grid=(NBLK,),
        in_specs=[
            pl.BlockSpec((BM, HID), lambda i: (i, 0)),
            _full2((1, HID)), _full2((1, HID)),
            _full2((1, HID)), _full2((1, HID)),
            _full2((HID, HID)), _full2((1, HID)),
            _full2((HID, 16)), _full2((16, HID)),
        ],
        out_specs=_PROJ_OUT_SPECS,
        out_shape=_PROJ_OUT_SHAPE,
    )(agg, s1, q1, bng, bnb, w, b, p, q)


def _combine_body(o0, o1, o2, o3, ot, r_ref, q_ref,
                  agg_ref, sum_ref, sq_ref, acc_s, acc_q):
    i = pl.program_id(0)
    t16 = (ot[0] + ot[1])[:, :16]
    r = r_ref[...]
    f = r / (r * t16 + 1e-12)
    fb = jnp.dot(f, q_ref[...], preferred_element_type=_f32)
    g = jnp.concatenate(
        [o0[0] + o0[1], o1[0] + o1[1], o2[0] + o2[1], o3[0] + o3[1]], axis=1)
    agg = g * fb
    agg_ref[...] = agg
    rowid = i * BM + lax.broadcasted_iota(jnp.int32, (BM, 1), 0)
    am = jnp.where(rowid < N, agg, 0.0)

    @pl.when(i == 0)
    def _():
        acc_s[...] = jnp.zeros_like(acc_s)
        acc_q[...] = jnp.zeros_like(acc_q)

    acc_s[...] += jnp.sum(am, axis=0, keepdims=True)
    acc_q[...] += jnp.sum(am * am, axis=0, keepdims=True)
    sum_ref[...] = acc_s[...]
    sq_ref[...] = acc_q[...]


def _combine(o0, o1, o2, o3, ot, r, q):
    return pl.pallas_call(
        _combine_body,
        grid=(NBLK,),
        in_specs=[
            pl.BlockSpec((NC, BM, 128), lambda i: (0, i, 0)),
            pl.BlockSpec((NC, BM, 128), lambda i: (0, i, 0)),
            pl.BlockSpec((NC, BM, 128), lambda i: (0, i, 0)),
            pl.BlockSpec((NC, BM, 128), lambda i: (0, i, 0)),
            pl.BlockSpec((NC, BM, 128), lambda i: (0, i, 0)),
            pl.BlockSpec((BM, 16), lambda i: (i, 0)),
            _full2((16, HID)),
        ],
        out_specs=[
            pl.BlockSpec((BM, HID), lambda i: (i, 0)),
            _full2((1, HID)), _full2((1, HID)),
        ],
        out_shape=[
            jax.ShapeDtypeStruct((NP, HID), _f32),
            jax.ShapeDtypeStruct((1, HID), _f32),
            jax.ShapeDtypeStruct((1, HID), _f32),
        ],
        scratch_shapes=[pltpu.VMEM((1, HID), _f32), pltpu.VMEM((1, HID), _f32)],
    )(o0, o1, o2, o3, ot, r, q)


def _head_body(x_ref, sum_ref, sq_ref, bng_ref, bnb_ref,
               w1_ref, b1_ref, w2_ref, b2_ref, out_ref):
    mean = sum_ref[...] / N
    var = sq_ref[...] / N - mean * mean
    scale = bng_ref[...] * lax.rsqrt(var + 1e-5)
    shift = bnb_ref[...] - mean * scale
    xb = jnp.maximum(x_ref[...] * scale + shift, 0.0)
    h1 = jnp.maximum(
        jnp.dot(xb, w1_ref[...], preferred_element_type=_f32) + b1_ref[...], 0.0)
    out_ref[...] = (
        jnp.dot(h1, w2_ref[...], preferred_element_type=_f32) + b2_ref[...])


def _head(agg, s2, q2, bng, bnb, w1, b1, w2, b2):
    return pl.pallas_call(
        _head_body,
        grid=(NBLK,),
        in_specs=[
            pl.BlockSpec((BM, HID), lambda i: (i, 0)),
            _full2((1, HID)), _full2((1, HID)),
            _full2((1, HID)), _full2((1, HID)),
            _full2((HID, MLP)), _full2((1, MLP)),
            _full2((MLP, OUT)), _full2((1, OUT)),
        ],
        out_specs=pl.BlockSpec((BM, OUT), lambda i: (i, 0)),
        out_shape=jax.ShapeDtypeStruct((NP, OUT), _f32),
    )(agg, s2, q2, bng, bnb, w1, b1, w2, b2)


def _sc_agg(packed, g0, g1, g2, g3, u, zg):
    mesh = plsc.VectorSubcoreMesh(core_axis_name="c", subcore_axis_name="s")

    scratch = [
        pltpu.VMEM_SHARED((NP, 128), _f32),
        pltpu.VMEM((NCHUNK, CH), jnp.int32),
    ]
    scratch += [pltpu.VMEM((CH,), jnp.int32) for _ in range(2 * PDEPTH)]
    scratch += [pltpu.VMEM((CH, 128), _f32) for _ in range(PDEPTH)]
    scratch += [pltpu.VMEM((ZR, 128), _f32)]
    scratch += [pltpu.SemaphoreType.DMA for _ in range(PDEPTH)]

    @functools.partial(
        pl.kernel,
        out_type=[jax.ShapeDtypeStruct((NC, NP, 128), _f32)] * 5,
        mesh=mesh,
        scratch_types=scratch,
    )
    def k(pk_h, g0_h, g1_h, g2_h, g3_h, u_h, zg_h,
          o0, o1, o2, o3, ot_h, accg, pkv, *bufs):
        srcs = bufs[0:2 * PDEPTH:2]
        dsts = bufs[1:2 * PDEPTH:2]
        rows = bufs[2 * PDEPTH:3 * PDEPTH]
        zbg = bufs[3 * PDEPTH]
        sems = bufs[3 * PDEPTH + 1:]
        sc = lax.axis_index("c")
        tid = lax.axis_index("s")
        wid = sc * NS + tid
        pltpu.sync_copy(pk_h.at[wid], pkv)
        pltpu.sync_copy(zg_h, zbg)
        gs = [u_h, g0_h, g1_h, g2_h, g3_h]
        os_ = [ot_h, o0, o1, o2, o3]
        row0 = tid * ROWS_T

        def unpack(c, srcb, dstb):
            for j in range(CH // 16):
                pk = pkv[c, pl.ds(j * 16, 16)]
                srcb[pl.ds(j * 16, 16)] = lax.bitwise_and(pk, 16383)
                dstb[pl.ds(j * 16, 16)] = lax.shift_right_logical(pk, 14)

        for p in range(5):
            g = gs[p]
            for j in range(ROWS_T // ZR):
                pltpu.sync_copy(zbg, accg.at[pl.ds(row0 + j * ZR, ZR)])
            plsc.subcore_barrier()

            srcb0, srcb1 = srcs[0], srcs[1]
            dstb0, dstb1 = dsts[0], dsts[1]
            rows0, rows1 = rows[0], rows[1]
            gs0, gs1 = sems[0], sems[1]
            unpack(0, srcb0, dstb0)
            pltpu.async_copy(g.at[srcb0], rows0, gs0)

            def body(kk, carry):
                c = kk * 2
                unpack(c + 1, srcb1, dstb1)
                pltpu.make_async_copy(g.at[srcb1], rows0, gs0).wait()
                pltpu.async_copy(g.at[srcb1], rows1, gs1)
                pltpu.sync_copy(rows0, accg.at[dstb0], add=True)
                unpack(c + 2, srcb0, dstb0)
                pltpu.make_async_copy(g.at[srcb0], rows1, gs1).wait()
                pltpu.async_copy(g.at[srcb0], rows0, gs0)
                pltpu.sync_copy(rows1, accg.at[dstb1], add=True)
                return carry

            lax.fori_loop(0, NCHUNK // 2 - 1, body, 0)
            unpack(NCHUNK - 1, srcb1, dstb1)
            pltpu.make_async_copy(g.at[srcb1], rows0, gs0).wait()
            pltpu.async_copy(g.at[srcb1], rows1, gs1)
            pltpu.sync_copy(rows0, accg.at[dstb0], add=True)
            pltpu.make_async_copy(g.at[srcb1], rows1, gs1).wait()
            pltpu.sync_copy(rows1, accg.at[dstb1], add=True)

            plsc.subcore_barrier()
            pltpu.sync_copy(accg.at[pl.ds(row0, ROWS_T)],
                            os_[p].at[sc, pl.ds(row0, ROWS_T)])

    return k(packed.reshape(NC * NS, NCHUNK, CH), g0, g1, g2, g3, u, zg)


def _pack_attn(a_l, a_r):
    eye = jnp.eye(HEADS, dtype=_f32)
    pl_ = (a_l[:, :, None] * eye[:, None, :]).reshape(HID, HEADS)
    pr_ = (a_r[:, :, None] * eye[:, None, :]).reshape(HID, HEADS)
    return jnp.concatenate([pl_, pr_], axis=1)


def kernel(x, edge_index, lin1_w, lin1_b, a_l1, a_r1, bn1_g, bn1_b,
           lin2_w, lin2_b, a_l2, a_r2, bn2_g, bn2_b,
           head_w1, head_b1, head_w2, head_b2):
    loops = jnp.arange(N, dtype=edge_index.dtype)
    ei = jnp.concatenate(
        [edge_index, edge_index[::-1], jnp.stack([loops, loops], axis=0)],
        axis=1)
    pad = EP - ET
    src = jnp.concatenate([ei[0], jnp.zeros((pad,), jnp.int32)])
    pad_dst = DUMMY_DST + (jnp.arange(pad, dtype=jnp.int32) % 128)
    dst = jnp.concatenate([ei[1], pad_dst])
    packed = jnp.left_shift(dst, 14) | src

    xp = jnp.zeros((NP, IN_DIM), _f32).at[:N].set(x)
    p1 = _pack_attn(a_l1, a_r1)
    p2 = _pack_attn(a_l2, a_r2)
    q = jnp.concatenate(
        [jnp.repeat(jnp.eye(HEADS, dtype=_f32), DPH, axis=1),
         jnp.zeros((HEADS, HID), _f32)], axis=0)
    zg = jnp.zeros((ZR, 128), _f32)
    row = lambda v: v.reshape(1, -1)

    g0, g1, g2, g3, u1, r1 = _proj1(xp, lin1_w, row(lin1_b), p1, q)
    o0, o1, o2, o3, ot = _sc_agg(packed, g0, g1, g2, g3, u1, zg)
    agg1, s1, q1 = _combine(o0, o1, o2, o3, ot, r1, q)

    g0, g1, g2, g3, u2, r2 = _proj2(agg1, s1, q1, row(bn1_g), row(bn1_b),
                                    lin2_w, row(lin2_b), p2, q)
    o0, o1, o2, o3, ot = _sc_agg(packed, g0, g1, g2, g3, u2, zg)
    agg2, s2, q2 = _combine(o0, o1, o2, o3, ot, r2, q)

    logits = _head(agg2, s2, q2, row(bn2_g), row(bn2_b),
                   head_w1, row(head_b1), head_w2, row(head_b2))
    return logits[:N]


# speedup vs baseline: 5.4581x; 2.8158x over previous
import functools

import jax
import jax.numpy as jnp
from jax import lax
from jax.experimental import pallas as pl
from jax.experimental.pallas import tpu as pltpu
from jax.experimental.pallas import tpu_sc as plsc

N = 10000
IN_DIM = 256
HID = 512
HEADS = 8
DPH = HID // HEADS
OUT = 128
MLP = 512

NP = 10240
BM = 256
NBLK = NP // BM
E_RAW = 160000
ET = 2 * E_RAW + N
NC, NS = 2, 16
CH = 128
NCHUNK = 82
PDEPTH = 2
E_TILE = CH * NCHUNK
E_SC = E_TILE * NS
EP = E_SC * NC
ROWS_T = NP // NS
ZR = 16
DUMMY_DST = N + 64

_f32 = jnp.float32


def _proj_body(x_ref, w_ref, b_ref, p_ref, q_ref,
               g0, g1, g2, g3, u_ref, r_ref):
    xb = x_ref[...]
    h = jnp.dot(xb, w_ref[...], preferred_element_type=_f32) + b_ref[...]
    s = jnp.dot(h, p_ref[...], preferred_element_type=_f32)
    e16 = jnp.exp(s)
    ub = jnp.dot(e16, q_ref[...], preferred_element_type=_f32)
    g = h * ub
    g0[...] = g[:, 0:128]
    g1[...] = g[:, 128:256]
    g2[...] = g[:, 256:384]
    g3[...] = g[:, 384:512]
    lane = lax.broadcasted_iota(jnp.int32, (BM, 16), 1)
    u_ref[...] = jnp.concatenate(
        [jnp.where(lane < HEADS, e16, 0.0), jnp.zeros((BM, 112), _f32)], axis=1)
    rh = e16[:, HEADS:]
    r_ref[...] = jnp.concatenate([rh, jnp.zeros_like(rh)], axis=1)


def _proj_bn_body(x_ref, sum_ref, sq_ref, bng_ref, bnb_ref,
                  w_ref, b_ref, p_ref, q_ref,
                  g0, g1, g2, g3, u_ref, r_ref):
    mean = sum_ref[...] / N
    var = sq_ref[...] / N - mean * mean
    scale = bng_ref[...] * lax.rsqrt(var + 1e-5)
    shift = bnb_ref[...] - mean * scale
    xb = jnp.maximum(x_ref[...] * scale + shift, 0.0)
    h = jnp.dot(xb, w_ref[...], preferred_element_type=_f32) + b_ref[...]
    s = jnp.dot(h, p_ref[...], preferred_element_type=_f32)
    e16 = jnp.exp(s)
    ub = jnp.dot(e16, q_ref[...], preferred_element_type=_f32)
    g = h * ub
    g0[...] = g[:, 0:128]
    g1[...] = g[:, 128:256]
    g2[...] = g[:, 256:384]
    g3[...] = g[:, 384:512]
    lane = lax.broadcasted_iota(jnp.int32, (BM, 16), 1)
    u_ref[...] = jnp.concatenate(
        [jnp.where(lane < HEADS, e16, 0.0), jnp.zeros((BM, 112), _f32)], axis=1)
    rh = e16[:, HEADS:]
    r_ref[...] = jnp.concatenate([rh, jnp.zeros_like(rh)], axis=1)


def _full2(shape):
    return pl.BlockSpec(shape, lambda i: (0, 0))


_PROJ_OUT_SPECS = (
    [pl.BlockSpec((BM, 128), lambda i: (i, 0))] * 5
    + [pl.BlockSpec((BM, 16), lambda i: (i, 0))]
)
_PROJ_OUT_SHAPE = (
    [jax.ShapeDtypeStruct((NP, 128), _f32)] * 5
    + [jax.ShapeDtypeStruct((NP, 16), _f32)]
)


def _proj1(xp, w, b, p, q):
    return pl.pallas_call(
        _proj_body,
        grid=(NBLK,),
        in_specs=[
            pl.BlockSpec((BM, IN_DIM), lambda i: (i, 0)),
            _full2((IN_DIM, HID)), _full2((1, HID)),
            _full2((HID, 16)), _full2((16, HID)),
        ],
        out_specs=_PROJ_OUT_SPECS,
        out_shape=_PROJ_OUT_SHAPE,
    )(xp, w, b, p, q)


def _proj2(agg, s1, q1, bng, bnb, w, b, p, q):
    return pl.pallas_call(
        _proj_bn_body,
        grid=(NBLK,),
        in_specs=[
            pl.BlockSpec((BM, HID), lambda i: (i, 0)),
            _full2((1, HID)), _full2((1, HID)),
            _full2((1, HID)), _full2((1, HID)),
            _full2((HID, HID)), _full2((1, HID)),
            _full2((HID, 16)), _full2((16, HID)),
        ],
        out_specs=_PROJ_OUT_SPECS,
        out_shape=_PROJ_OUT_SHAPE,
    )(agg, s1, q1, bng, bnb, w, b, p, q)


def _combine_body(o0, o1, o2, o3, ot, r_ref, q_ref,
                  agg_ref, sum_ref, sq_ref, acc_s, acc_q):
    i = pl.program_id(0)
    t16 = (ot[0] + ot[1])[:, :16]
    r = r_ref[...]
    f = r / (r * t16 + 1e-12)
    fb = jnp.dot(f, q_ref[...], preferred_element_type=_f32)
    g = jnp.concatenate(
        [o0[0] + o0[1], o1[0] + o1[1], o2[0] + o2[1], o3[0] + o3[1]], axis=1)
    agg = g * fb
    agg_ref[...] = agg
    rowid = i * BM + lax.broadcasted_iota(jnp.int32, (BM, 1), 0)
    am = jnp.where(rowid < N, agg, 0.0)

    @pl.when(i == 0)
    def _():
        acc_s[...] = jnp.zeros_like(acc_s)
        acc_q[...] = jnp.zeros_like(acc_q)

    acc_s[...] += jnp.sum(am, axis=0, keepdims=True)
    acc_q[...] += jnp.sum(am * am, axis=0, keepdims=True)
    sum_ref[...] = acc_s[...]
    sq_ref[...] = acc_q[...]


def _combine(o0, o1, o2, o3, ot, r, q):
    return pl.pallas_call(
        _combine_body,
        grid=(NBLK,),
        in_specs=[
            pl.BlockSpec((NC, BM, 128), lambda i: (0, i, 0)),
            pl.BlockSpec((NC, BM, 128), lambda i: (0, i, 0)),
            pl.BlockSpec((NC, BM, 128), lambda i: (0, i, 0)),
            pl.BlockSpec((NC, BM, 128), lambda i: (0, i, 0)),
            pl.BlockSpec((NC, BM, 128), lambda i: (0, i, 0)),
            pl.BlockSpec((BM, 16), lambda i: (i, 0)),
            _full2((16, HID)),
        ],
        out_specs=[
            pl.BlockSpec((BM, HID), lambda i: (i, 0)),
            _full2((1, HID)), _full2((1, HID)),
        ],
        out_shape=[
            jax.ShapeDtypeStruct((NP, HID), _f32),
            jax.ShapeDtypeStruct((1, HID), _f32),
            jax.ShapeDtypeStruct((1, HID), _f32),
        ],
        scratch_shapes=[pltpu.VMEM((1, HID), _f32), pltpu.VMEM((1, HID), _f32)],
    )(o0, o1, o2, o3, ot, r, q)


def _head_body(x_ref, sum_ref, sq_ref, bng_ref, bnb_ref,
               w1_ref, b1_ref, w2_ref, b2_ref, out_ref):
    mean = sum_ref[...] / N
    var = sq_ref[...] / N - mean * mean
    scale = bng_ref[...] * lax.rsqrt(var + 1e-5)
    shift = bnb_ref[...] - mean * scale
    xb = jnp.maximum(x_ref[...] * scale + shift, 0.0)
    h1 = jnp.maximum(
        jnp.dot(xb, w1_ref[...], preferred_element_type=_f32) + b1_ref[...], 0.0)
    out_ref[...] = (
        jnp.dot(h1, w2_ref[...], preferred_element_type=_f32) + b2_ref[...])


def _head(agg, s2, q2, bng, bnb, w1, b1, w2, b2):
    return pl.pallas_call(
        _head_body,
        grid=(NBLK,),
        in_specs=[
            pl.BlockSpec((BM, HID), lambda i: (i, 0)),
            _full2((1, HID)), _full2((1, HID)),
            _full2((1, HID)), _full2((1, HID)),
            _full2((HID, MLP)), _full2((1, MLP)),
            _full2((MLP, OUT)), _full2((1, OUT)),
        ],
        out_specs=pl.BlockSpec((BM, OUT), lambda i: (i, 0)),
        out_shape=jax.ShapeDtypeStruct((NP, OUT), _f32),
    )(agg, s2, q2, bng, bnb, w1, b1, w2, b2)


def _sc_agg(packed, g0, g1, g2, g3, u, zg):
    mesh = plsc.VectorSubcoreMesh(core_axis_name="c", subcore_axis_name="s")

    scratch = [
        pltpu.VMEM_SHARED((NP, 128), _f32),
        pltpu.VMEM((NCHUNK, CH), jnp.int32),
    ]
    scratch += [pltpu.VMEM((CH,), jnp.int32) for _ in range(2 * PDEPTH)]
    scratch += [pltpu.VMEM((CH, 128), _f32) for _ in range(PDEPTH)]
    scratch += [pltpu.VMEM((ZR, 128), _f32)]
    scratch += [pltpu.SemaphoreType.DMA for _ in range(PDEPTH)]

    @functools.partial(
        pl.kernel,
        out_type=[jax.ShapeDtypeStruct((NC, NP, 128), _f32)] * 5,
        mesh=mesh,
        scratch_types=scratch,
    )
    def k(pk_h, g0_h, g1_h, g2_h, g3_h, u_h, zg_h,
          o0, o1, o2, o3, ot_h, accg, pkv, *bufs):
        srcs = bufs[0:2 * PDEPTH:2]
        dsts = bufs[1:2 * PDEPTH:2]
        rows = bufs[2 * PDEPTH:3 * PDEPTH]
        zbg = bufs[3 * PDEPTH]
        sems = bufs[3 * PDEPTH + 1:]
        sc = lax.axis_index("c")
        tid = lax.axis_index("s")
        wid = sc * NS + tid
        pltpu.sync_copy(pk_h.at[wid], pkv)
        pltpu.sync_copy(zg_h, zbg)
        gs = [u_h, g0_h, g1_h, g2_h, g3_h]
        os_ = [ot_h, o0, o1, o2, o3]
        row0 = tid * ROWS_T

        def unpack(c, srcb, dstb):
            for j in range(CH // 16):
                pk = pkv[c, pl.ds(j * 16, 16)]
                srcb[pl.ds(j * 16, 16)] = lax.bitwise_and(pk, 16383)
                dstb[pl.ds(j * 16, 16)] = lax.shift_right_logical(pk, 14)

        for p in range(5):
            g = gs[p]
            for j in range(ROWS_T // ZR):
                pltpu.sync_copy(zbg, accg.at[pl.ds(row0 + j * ZR, ZR)])
            plsc.subcore_barrier()

            srcb0, srcb1 = srcs[0], srcs[1]
            dstb0, dstb1 = dsts[0], dsts[1]
            rows0, rows1 = rows[0], rows[1]
            gs0, gs1 = sems[0], sems[1]
            unpack(0, srcb0, dstb0)
            pltpu.async_copy(g.at[srcb0], rows0, gs0)

            def body(kk, carry):
                c = kk * 2
                unpack(c + 1, srcb1, dstb1)
                pltpu.make_async_copy(g.at[srcb1], rows0, gs0).wait()
                pltpu.async_copy(g.at[srcb1], rows1, gs1)
                pltpu.sync_copy(rows0, accg.at[dstb0], add=True)
                unpack(c + 2, srcb0, dstb0)
                pltpu.make_async_copy(g.at[srcb0], rows1, gs1).wait()
                pltpu.async_copy(g.at[srcb0], rows0, gs0)
                pltpu.sync_copy(rows1, accg.at[dstb1], add=True)
                return carry

            lax.fori_loop(0, NCHUNK // 2 - 1, body, 0)
            unpack(NCHUNK - 1, srcb1, dstb1)
            pltpu.make_async_copy(g.at[srcb1], rows0, gs0).wait()
            pltpu.async_copy(g.at[srcb1], rows1, gs1)
            pltpu.sync_copy(rows0, accg.at[dstb0], add=True)
            pltpu.make_async_copy(g.at[srcb1], rows1, gs1).wait()
            pltpu.sync_copy(rows1, accg.at[dstb1], add=True)

            plsc.subcore_barrier()
            pltpu.sync_copy(accg.at[pl.ds(row0, ROWS_T)],
                            os_[p].at[sc, pl.ds(row0, ROWS_T)])

    return k(packed.reshape(NC * NS, NCHUNK, CH), g0, g1, g2, g3, u, zg)


def _pack_attn(a_l, a_r):
    eye = jnp.eye(HEADS, dtype=_f32)
    pl_ = (a_l[:, :, None] * eye[:, None, :]).reshape(HID, HEADS)
    pr_ = (a_r[:, :, None] * eye[:, None, :]).reshape(HID, HEADS)
    return jnp.concatenate([pl_, pr_], axis=1)


def kernel(x, edge_index, lin1_w, lin1_b, a_l1, a_r1, bn1_g, bn1_b,
           lin2_w, lin2_b, a_l2, a_r2, bn2_g, bn2_b,
           head_w1, head_b1, head_w2, head_b2):
    loops = jnp.arange(N, dtype=edge_index.dtype)
    ei = jnp.concatenate(
        [edge_index, edge_index[::-1], jnp.stack([loops, loops], axis=0)],
        axis=1)
    pad = EP - ET
    pad_src = jnp.arange(pad, dtype=jnp.int32) % N
    src = jnp.concatenate([ei[0], pad_src])
    pad_dst = DUMMY_DST + (jnp.arange(pad, dtype=jnp.int32) % 128)
    dst = jnp.concatenate([ei[1], pad_dst])
    packed = jnp.left_shift(dst, 14) | src

    xp = jnp.zeros((NP, IN_DIM), _f32).at[:N].set(x)
    p1 = _pack_attn(a_l1, a_r1)
    p2 = _pack_attn(a_l2, a_r2)
    q = jnp.concatenate(
        [jnp.repeat(jnp.eye(HEADS, dtype=_f32), DPH, axis=1),
         jnp.zeros((HEADS, HID), _f32)], axis=0)
    zg = jnp.zeros((ZR, 128), _f32)
    row = lambda v: v.reshape(1, -1)

    g0, g1, g2, g3, u1, r1 = _proj1(xp, lin1_w, row(lin1_b), p1, q)
    o0, o1, o2, o3, ot = _sc_agg(packed, g0, g1, g2, g3, u1, zg)
    agg1, s1, q1 = _combine(o0, o1, o2, o3, ot, r1, q)

    g0, g1, g2, g3, u2, r2 = _proj2(agg1, s1, q1, row(bn1_g), row(bn1_b),
                                    lin2_w, row(lin2_b), p2, q)
    o0, o1, o2, o3, ot = _sc_agg(packed, g0, g1, g2, g3, u2, zg)
    agg2, s2, q2 = _combine(o0, o1, o2, o3, ot, r2, q)

    logits = _head(agg2, s2, q2, row(bn2_g), row(bn2_b),
                   head_w1, row(head_b1), head_w2, row(head_b2))
    return logits[:N]
